# baseline (speedup 1.0000x reference)
import functools

import jax
import jax.numpy as jnp
from jax import lax
from jax.experimental import pallas as pl
from jax.experimental.pallas import tpu as pltpu
from jax.experimental.pallas import tpu_sc as plsc

N = 4194304
NC = 2
NS = 16
NW = NC * NS
PER_W = N // NW
C = 4096
NCHUNK = PER_W // C
NPAIR = NCHUNK // 2
UNROLL = 8
ROWS = C // 512

_GDN = lax.GatherDimensionNumbers(
    offset_dims=(), collapsed_slice_dims=(0,), start_index_map=(0,))


def _take16(vec, idx):
    return lax.gather(vec, idx.reshape(16, 1), _GDN, slice_sizes=(1,),
                      mode=lax.GatherScatterMode.PROMISE_IN_BOUNDS)


def _bcast(vec, lane):
    return _take16(vec, jnp.full((16,), lane, jnp.int32))


def _body(sup_h, cap_h, tgt_h, rnd_h, tbl_h,
          capo_h, supo_h,
          sup_v, cap_v, r0_v, r1_v, tgt_v, capo_v, supo_v,
          tbl_v, in_sem, out_sem):
    wid = lax.axis_index("s") * NC + lax.axis_index("c")
    base_w = wid * PER_W

    pltpu.sync_copy(tbl_h, tbl_v)
    tbl = tbl_v[...]
    v0, v1, v2, v3 = (_bcast(tbl, j) for j in range(4))
    b0, b1, b2 = (_bcast(tbl, 4 + j) for j in range(3))
    tgt_w = tgt_h.bitcast(jnp.int32)

    def start_in(ci, b):
        base = pl.multiple_of(base_w + ci * C, 4096)
        pltpu.async_copy(sup_h.at[pl.ds(base, C)], sup_v[b], in_sem[b])
        pltpu.async_copy(cap_h.at[pl.ds(base, C)], cap_v[b], in_sem[b])
        pltpu.async_copy(rnd_h.at[0, pl.ds(base, C)], r0_v[b], in_sem[b])
        pltpu.async_copy(rnd_h.at[1, pl.ds(base, C)], r1_v[b], in_sem[b])
        rbase = pl.multiple_of(base // 512, ROWS)
        pltpu.async_copy(tgt_w.at[pl.ds(rbase, ROWS), :], tgt_v[b], in_sem[b])

    def wait_in(b):
        pltpu.make_async_copy(sup_h.at[pl.ds(0, C)], sup_v[b], in_sem[b]).wait()
        pltpu.make_async_copy(cap_h.at[pl.ds(0, C)], cap_v[b], in_sem[b]).wait()
        pltpu.make_async_copy(rnd_h.at[0, pl.ds(0, C)], r0_v[b], in_sem[b]).wait()
        pltpu.make_async_copy(rnd_h.at[1, pl.ds(0, C)], r1_v[b], in_sem[b]).wait()
        pltpu.make_async_copy(tgt_w.at[pl.ds(0, ROWS), :], tgt_v[b],
                              in_sem[b]).wait()

    def start_out(ci, b):
        base = pl.multiple_of(base_w + ci * C, 4096)
        pltpu.async_copy(capo_v[b], capo_h.at[pl.ds(base, C)], out_sem[b])
        pltpu.async_copy(supo_v[b], supo_h.at[pl.ds(base, C)], out_sem[b])

    def wait_out(b):
        pltpu.make_async_copy(capo_v[b], capo_h.at[pl.ds(0, C)], out_sem[b]).wait()
        pltpu.make_async_copy(supo_v[b], supo_h.at[pl.ds(0, C)], out_sem[b]).wait()

    def compute(b):
        def row(g, c2):
            i = lax.shift_right_logical(g, 3)
            jq = g & 7
            if True:
                j0 = jq * 16
                w = tgt_v[b][i, pl.ds(j0, 16)]
                for p in range(4):
                    off = i * 512 + p * 128 + j0
                    sl = pl.ds(off, 16)
                    t = (lax.shift_right_logical(w, 8 * p) & 1) != 0
                    r0 = r0_v[b][sl]
                    r1 = r1_v[b][sl]
                    sup = sup_v[b][sl]
                    cap = cap_v[b][sl]
                    nm = jnp.where(b1 < r0,
                                   jnp.where(b2 < r0, v3, v2),
                                   jnp.where(b0 < r0, v1, v0))
                    sw = t & (r1 < 0.5)
                    capo_v[b][sl] = jnp.where(sw, nm, cap)
                    supo_v[b][sl] = jnp.where(sw, nm + (sup - cap), sup)
            return c2
        lax.fori_loop(0, ROWS * 8, row, 0)

    start_in(0, 0)

    def pair_body(i, carry):
        ci_a = 2 * i
        ci_b = 2 * i + 1
        start_in(ci_b, 1)
        wait_in(0)

        @pl.when(i > 0)
        def _():
            wait_out(0)
        compute(0)
        start_out(ci_a, 0)

        @pl.when(i + 1 < NPAIR)
        def _():
            start_in(ci_a + 2, 0)
        wait_in(1)

        @pl.when(i > 0)
        def _():
            wait_out(1)
        compute(1)
        start_out(ci_b, 1)
        return carry

    lax.fori_loop(0, NPAIR, pair_body, 0)
    wait_out(0)
    wait_out(1)


@jax.jit
def kernel(suppressants, capacity, targets, randomness_source,
           possible_capacities, cum_probs):
    tgt_u8 = targets.astype(jnp.uint8).reshape(N // 128, 128)
    tbl = jnp.concatenate([
        possible_capacities.astype(jnp.float32),
        cum_probs.astype(jnp.float32),
        jnp.zeros((8,), jnp.float32),
    ])
    f32 = jnp.float32
    vbuf = lambda dt: (pltpu.VMEM((C,), dt), pltpu.VMEM((C,), dt))
    run = pl.kernel(
        _body,
        out_type=(jax.ShapeDtypeStruct((N,), f32),
                  jax.ShapeDtypeStruct((N,), f32)),
        mesh=plsc.VectorSubcoreMesh(core_axis_name="c", subcore_axis_name="s"),
        scratch_types=[
            vbuf(f32),
            vbuf(f32),
            vbuf(f32),
            vbuf(f32),
            (pltpu.VMEM((ROWS, 128), jnp.int32),
             pltpu.VMEM((ROWS, 128), jnp.int32)),
            vbuf(f32),
            vbuf(f32),
            pltpu.VMEM((16,), f32),
            (pltpu.SemaphoreType.DMA, pltpu.SemaphoreType.DMA),
            (pltpu.SemaphoreType.DMA, pltpu.SemaphoreType.DMA),
        ],
    )
    capacity_new, suppressants_new = run(
        suppressants, capacity, tgt_u8, randomness_source, tbl)
    return capacity_new, suppressants_new


# speedup vs baseline: 1.0493x; 1.0493x over previous
import functools

import jax
import jax.numpy as jnp
from jax import lax
from jax.experimental import pallas as pl
from jax.experimental.pallas import tpu as pltpu
from jax.experimental.pallas import tpu_sc as plsc

N = 4194304
NC = 2
NS = 16
NW = NC * NS
PER_W = N // NW
C = 8192
NCHUNK = PER_W // C
NPAIR = NCHUNK // 2
UNROLL = 8
ROWS = C // 512

_GDN = lax.GatherDimensionNumbers(
    offset_dims=(), collapsed_slice_dims=(0,), start_index_map=(0,))


def _take16(vec, idx):
    return lax.gather(vec, idx.reshape(16, 1), _GDN, slice_sizes=(1,),
                      mode=lax.GatherScatterMode.PROMISE_IN_BOUNDS)


def _bcast(vec, lane):
    return _take16(vec, jnp.full((16,), lane, jnp.int32))


def _body(sup_h, cap_h, tgt_h, rnd_h, tbl_h,
          capo_h, supo_h,
          sup_v, cap_v, r0_v, r1_v, tgt_v, capo_v, supo_v,
          tbl_v, in_sem, out_sem):
    wid = lax.axis_index("s") * NC + lax.axis_index("c")
    base_w = wid * PER_W

    pltpu.sync_copy(tbl_h, tbl_v)
    tbl = tbl_v[...]
    v0, v1, v2, v3 = (_bcast(tbl, j) for j in range(4))
    b0, b1, b2 = (_bcast(tbl, 4 + j) for j in range(3))
    tgt_w = tgt_h.bitcast(jnp.int32)

    def start_in(ci, b):
        base = pl.multiple_of(base_w + ci * C, 4096)
        pltpu.async_copy(sup_h.at[pl.ds(base, C)], sup_v[b], in_sem[b])
        pltpu.async_copy(cap_h.at[pl.ds(base, C)], cap_v[b], in_sem[b])
        pltpu.async_copy(rnd_h.at[0, pl.ds(base, C)], r0_v[b], in_sem[b])
        pltpu.async_copy(rnd_h.at[1, pl.ds(base, C)], r1_v[b], in_sem[b])
        rbase = pl.multiple_of(base // 512, ROWS)
        pltpu.async_copy(tgt_w.at[pl.ds(rbase, ROWS), :], tgt_v[b], in_sem[b])

    def wait_in(b):
        pltpu.make_async_copy(sup_h.at[pl.ds(0, C)], sup_v[b], in_sem[b]).wait()
        pltpu.make_async_copy(cap_h.at[pl.ds(0, C)], cap_v[b], in_sem[b]).wait()
        pltpu.make_async_copy(rnd_h.at[0, pl.ds(0, C)], r0_v[b], in_sem[b]).wait()
        pltpu.make_async_copy(rnd_h.at[1, pl.ds(0, C)], r1_v[b], in_sem[b]).wait()
        pltpu.make_async_copy(tgt_w.at[pl.ds(0, ROWS), :], tgt_v[b],
                              in_sem[b]).wait()

    def start_out(ci, b):
        base = pl.multiple_of(base_w + ci * C, 4096)
        pltpu.async_copy(capo_v[b], capo_h.at[pl.ds(base, C)], out_sem[b])
        pltpu.async_copy(supo_v[b], supo_h.at[pl.ds(base, C)], out_sem[b])

    def wait_out(b):
        pltpu.make_async_copy(capo_v[b], capo_h.at[pl.ds(0, C)], out_sem[b]).wait()
        pltpu.make_async_copy(supo_v[b], supo_h.at[pl.ds(0, C)], out_sem[b]).wait()

    def compute(b):
        def row(g, c2):
            i = lax.shift_right_logical(g, 3)
            jq = g & 7
            if True:
                j0 = jq * 16
                w = tgt_v[b][i, pl.ds(j0, 16)]
                for p in range(4):
                    off = i * 512 + p * 128 + j0
                    sl = pl.ds(off, 16)
                    t = (lax.shift_right_logical(w, 8 * p) & 1) != 0
                    r0 = r0_v[b][sl]
                    r1 = r1_v[b][sl]
                    sup = sup_v[b][sl]
                    cap = cap_v[b][sl]
                    nm = jnp.where(b1 < r0,
                                   jnp.where(b2 < r0, v3, v2),
                                   jnp.where(b0 < r0, v1, v0))
                    sw = t & (r1 < 0.5)
                    capo_v[b][sl] = jnp.where(sw, nm, cap)
                    supo_v[b][sl] = jnp.where(sw, nm + (sup - cap), sup)
            return c2
        lax.fori_loop(0, ROWS * 8, row, 0)

    start_in(0, 0)

    def pair_body(i, carry):
        ci_a = 2 * i
        ci_b = 2 * i + 1
        start_in(ci_b, 1)
        wait_in(0)

        @pl.when(i > 0)
        def _():
            wait_out(0)
        compute(0)
        start_out(ci_a, 0)

        @pl.when(i + 1 < NPAIR)
        def _():
            start_in(ci_a + 2, 0)
        wait_in(1)

        @pl.when(i > 0)
        def _():
            wait_out(1)
        compute(1)
        start_out(ci_b, 1)
        return carry

    lax.fori_loop(0, NPAIR, pair_body, 0)
    wait_out(0)
    wait_out(1)


@jax.jit
def kernel(suppressants, capacity, targets, randomness_source,
           possible_capacities, cum_probs):
    tgt_u8 = targets.astype(jnp.uint8).reshape(N // 128, 128)
    tbl = jnp.concatenate([
        possible_capacities.astype(jnp.float32),
        cum_probs.astype(jnp.float32),
        jnp.zeros((8,), jnp.float32),
    ])
    f32 = jnp.float32
    vbuf = lambda dt: (pltpu.VMEM((C,), dt), pltpu.VMEM((C,), dt))
    run = pl.kernel(
        _body,
        out_type=(jax.ShapeDtypeStruct((N,), f32),
                  jax.ShapeDtypeStruct((N,), f32)),
        mesh=plsc.VectorSubcoreMesh(core_axis_name="c", subcore_axis_name="s"),
        scratch_types=[
            vbuf(f32),
            vbuf(f32),
            vbuf(f32),
            vbuf(f32),
            (pltpu.VMEM((ROWS, 128), jnp.int32),
             pltpu.VMEM((ROWS, 128), jnp.int32)),
            vbuf(f32),
            vbuf(f32),
            pltpu.VMEM((16,), f32),
            (pltpu.SemaphoreType.DMA, pltpu.SemaphoreType.DMA),
            (pltpu.SemaphoreType.DMA, pltpu.SemaphoreType.DMA),
        ],
    )
    capacity_new, suppressants_new = run(
        suppressants, capacity, tgt_u8, randomness_source, tbl)
    return capacity_new, suppressants_new


# speedup vs baseline: 1.0521x; 1.0026x over previous
import jax
import jax.numpy as jnp
from jax import lax
from jax.experimental import pallas as pl
from jax.experimental.pallas import tpu as pltpu
from jax.experimental.pallas import tpu_sc as plsc

N = 4194304
NC = 2
NS = 16
NW = NC * NS
PER_W = N // NW
C = 8192
NCHUNK = PER_W // C
NPAIR = NCHUNK // 2
ROWS = C // 512

_GDN = lax.GatherDimensionNumbers(
    offset_dims=(), collapsed_slice_dims=(0,), start_index_map=(0,))


def _take16(vec, idx):
    return lax.gather(vec, idx.reshape(16, 1), _GDN, slice_sizes=(1,),
                      mode=lax.GatherScatterMode.PROMISE_IN_BOUNDS)


def _bcast(vec, lane):
    return _take16(vec, jnp.full((16,), lane, jnp.int32))


def _body(sup_h, cap_h, tgt_h, rnd_h, tbl_h,
          capo_h, supo_h,
          sup_v, cap_v, r0_v, r1_v, tgt_v, capo_v, supo_v,
          tbl_v, in_sem, out_sem):
    wid = lax.axis_index("s") * NC + lax.axis_index("c")
    base_w = wid * PER_W

    pltpu.sync_copy(tbl_h, tbl_v)
    tbl = tbl_v[...]
    v0, v1, v2, v3 = (_bcast(tbl, j) for j in range(4))
    b0, b1, b2 = (_bcast(tbl, 4 + j) for j in range(3))
    tgt_w = tgt_h.bitcast(jnp.int32)

    def start_in(ci, b):
        base = pl.multiple_of(base_w + ci * C, 4096)
        pltpu.async_copy(sup_h.at[pl.ds(base, C)], sup_v[b], in_sem[b])
        pltpu.async_copy(cap_h.at[pl.ds(base, C)], cap_v[b], in_sem[b])
        pltpu.async_copy(rnd_h.at[0, pl.ds(base, C)], r0_v[b], in_sem[b])
        pltpu.async_copy(rnd_h.at[1, pl.ds(base, C)], r1_v[b], in_sem[b])
        rbase = pl.multiple_of(base // 512, ROWS)
        pltpu.async_copy(tgt_w.at[pl.ds(rbase, ROWS), :], tgt_v[b], in_sem[b])

    def wait_in(b):
        pltpu.make_async_copy(sup_h.at[pl.ds(0, C)], sup_v[b], in_sem[b]).wait()
        pltpu.make_async_copy(cap_h.at[pl.ds(0, C)], cap_v[b], in_sem[b]).wait()
        pltpu.make_async_copy(rnd_h.at[0, pl.ds(0, C)], r0_v[b], in_sem[b]).wait()
        pltpu.make_async_copy(rnd_h.at[1, pl.ds(0, C)], r1_v[b], in_sem[b]).wait()
        pltpu.make_async_copy(tgt_w.at[pl.ds(0, ROWS), :], tgt_v[b],
                              in_sem[b]).wait()

    def start_out(ci, b):
        base = pl.multiple_of(base_w + ci * C, 4096)
        pltpu.async_copy(capo_v[b], capo_h.at[pl.ds(base, C)], out_sem[b])
        pltpu.async_copy(supo_v[b], supo_h.at[pl.ds(base, C)], out_sem[b])

    def wait_out(b):
        pltpu.make_async_copy(capo_v[b], capo_h.at[pl.ds(0, C)], out_sem[b]).wait()
        pltpu.make_async_copy(supo_v[b], supo_h.at[pl.ds(0, C)], out_sem[b]).wait()

    def compute(b):
        def row(g, c2):
            i = lax.shift_right_logical(g, 3)
            j0 = (g & 7) * 16
            w = tgt_v[b][i, pl.ds(j0, 16)]
            for p in range(4):
                off = i * 512 + p * 128 + j0
                sl = pl.ds(off, 16)
                t = (lax.shift_right_logical(w, 8 * p) & 1) != 0
                r0 = r0_v[b][sl]
                r1 = r1_v[b][sl]
                sup = sup_v[b][sl]
                cap = cap_v[b][sl]
                nm = jnp.where(b1 < r0,
                               jnp.where(b2 < r0, v3, v2),
                               jnp.where(b0 < r0, v1, v0))
                sw = t & (r1 < 0.5)
                capo_v[b][sl] = jnp.where(sw, nm, cap)
                supo_v[b][sl] = jnp.where(sw, nm + (sup - cap), sup)
            return c2
        lax.fori_loop(0, ROWS * 8, row, 0)

    start_in(0, 0)

    def pair_body(i, carry):
        ci_a = 2 * i
        ci_b = 2 * i + 1
        start_in(ci_b, 1)
        wait_in(0)

        @pl.when(i > 0)
        def _():
            wait_out(0)
        compute(0)
        start_out(ci_a, 0)

        @pl.when(i + 1 < NPAIR)
        def _():
            start_in(ci_a + 2, 0)
        wait_in(1)

        @pl.when(i > 0)
        def _():
            wait_out(1)
        compute(1)
        start_out(ci_b, 1)
        return carry

    lax.fori_loop(0, NPAIR, pair_body, 0)
    wait_out(0)
    wait_out(1)


@jax.jit
def kernel(suppressants, capacity, targets, randomness_source,
           possible_capacities, cum_probs):
    tgt_u8 = targets.astype(jnp.uint8).reshape(N // 128, 128)
    tbl = jnp.concatenate([
        possible_capacities.astype(jnp.float32),
        cum_probs.astype(jnp.float32),
        jnp.zeros((8,), jnp.float32),
    ])
    f32 = jnp.float32
    vbuf = lambda dt: (pltpu.VMEM((C,), dt), pltpu.VMEM((C,), dt))
    run = pl.kernel(
        _body,
        out_type=(jax.ShapeDtypeStruct((N,), f32),
                  jax.ShapeDtypeStruct((N,), f32)),
        mesh=plsc.VectorSubcoreMesh(core_axis_name="c", subcore_axis_name="s"),
        scratch_types=[
            vbuf(f32),
            vbuf(f32),
            vbuf(f32),
            vbuf(f32),
            (pltpu.VMEM((ROWS, 128), jnp.int32),
             pltpu.VMEM((ROWS, 128), jnp.int32)),
            vbuf(f32),
            vbuf(f32),
            pltpu.VMEM((16,), f32),
            (pltpu.SemaphoreType.DMA, pltpu.SemaphoreType.DMA),
            (pltpu.SemaphoreType.DMA, pltpu.SemaphoreType.DMA),
        ],
    )
    capacity_new, suppressants_new = run(
        suppressants, capacity, tgt_u8, randomness_source, tbl)
    return capacity_new, suppressants_new


# speedup vs baseline: 1.0615x; 1.0089x over previous
import jax
import jax.numpy as jnp
from jax import lax
from jax.experimental import pallas as pl
from jax.experimental.pallas import tpu as pltpu
from jax.experimental.pallas import tpu_sc as plsc

N = 4194304
NC = 2
NS = 16
NW = NC * NS
PER_W = N // NW
C = 8192
NCHUNK = PER_W // C
NPAIR = NCHUNK // 2
ROWS = C // 512

_GDN = lax.GatherDimensionNumbers(
    offset_dims=(), collapsed_slice_dims=(0,), start_index_map=(0,))


def _take16(vec, idx):
    return lax.gather(vec, idx.reshape(16, 1), _GDN, slice_sizes=(1,),
                      mode=lax.GatherScatterMode.PROMISE_IN_BOUNDS)


def _bcast(vec, lane):
    return _take16(vec, jnp.full((16,), lane, jnp.int32))


def _body(sup_h, cap_h, tgt_h, rnd_h, tbl_h,
          capo_h, supo_h,
          sup_v, cap_v, r0_v, r1_v, tgt_v, capo_v, supo_v,
          tbl_v, in_sem, out_sem):
    wid = lax.axis_index("s") * NC + lax.axis_index("c")
    base_w = wid * PER_W
    tgt_w = tgt_h.bitcast(jnp.int32)

    def start_in(ci, b):
        base = pl.multiple_of(base_w + ci * C, 4096)
        pltpu.async_copy(sup_h.at[pl.ds(base, C)], sup_v[b], in_sem[b])
        pltpu.async_copy(cap_h.at[pl.ds(base, C)], cap_v[b], in_sem[b])
        pltpu.async_copy(rnd_h.at[0, pl.ds(base, C)], r0_v[b], in_sem[b])
        pltpu.async_copy(rnd_h.at[1, pl.ds(base, C)], r1_v[b], in_sem[b])
        rbase = pl.multiple_of(base // 512, ROWS)
        pltpu.async_copy(tgt_w.at[pl.ds(rbase, ROWS), :], tgt_v[b], in_sem[b])

    def wait_in(b):
        pltpu.make_async_copy(sup_h.at[pl.ds(0, C)], sup_v[b], in_sem[b]).wait()
        pltpu.make_async_copy(cap_h.at[pl.ds(0, C)], cap_v[b], in_sem[b]).wait()
        pltpu.make_async_copy(rnd_h.at[0, pl.ds(0, C)], r0_v[b], in_sem[b]).wait()
        pltpu.make_async_copy(rnd_h.at[1, pl.ds(0, C)], r1_v[b], in_sem[b]).wait()
        pltpu.make_async_copy(tgt_w.at[pl.ds(0, ROWS), :], tgt_v[b],
                              in_sem[b]).wait()

    def start_out(ci, b):
        base = pl.multiple_of(base_w + ci * C, 4096)
        pltpu.async_copy(capo_v[b], capo_h.at[pl.ds(base, C)], out_sem[b])
        pltpu.async_copy(supo_v[b], supo_h.at[pl.ds(base, C)], out_sem[b])

    def wait_out(b):
        pltpu.make_async_copy(capo_v[b], capo_h.at[pl.ds(0, C)], out_sem[b]).wait()
        pltpu.make_async_copy(supo_v[b], supo_h.at[pl.ds(0, C)], out_sem[b]).wait()

    def compute(b):
        def row(g, c2):
            i = lax.shift_right_logical(g, 3)
            j0 = (g & 7) * 16
            w = tgt_v[b][i, pl.ds(j0, 16)]
            for p in range(4):
                off = i * 512 + p * 128 + j0
                sl = pl.ds(off, 16)
                t = (lax.shift_right_logical(w, 8 * p) & 1) != 0
                r0 = r0_v[b][sl]
                r1 = r1_v[b][sl]
                sup = sup_v[b][sl]
                cap = cap_v[b][sl]
                nm = jnp.where(b1 < r0,
                               jnp.where(b2 < r0, v3, v2),
                               jnp.where(b0 < r0, v1, v0))
                sw = t & (r1 < 0.5)
                capo_v[b][sl] = jnp.where(sw, nm, cap)
                supo_v[b][sl] = jnp.where(sw, nm + (sup - cap), sup)
            return c2
        lax.fori_loop(0, ROWS * 8, row, 0)

    start_in(0, 0)
    start_in(1, 1)

    pltpu.sync_copy(tbl_h, tbl_v)
    tbl = tbl_v[...]
    v0, v1, v2, v3 = (_bcast(tbl, j) for j in range(4))
    b0, b1, b2 = (_bcast(tbl, 4 + j) for j in range(3))

    def pair_body(i, carry):
        ci_a = 2 * i
        ci_b = 2 * i + 1
        wait_in(0)

        @pl.when(i > 0)
        def _():
            wait_out(0)
        compute(0)
        start_out(ci_a, 0)

        @pl.when(i + 1 < NPAIR)
        def _():
            start_in(ci_a + 2, 0)
        wait_in(1)

        @pl.when(i > 0)
        def _():
            wait_out(1)
        compute(1)
        start_out(ci_b, 1)

        @pl.when(i + 1 < NPAIR)
        def _():
            start_in(ci_b + 2, 1)
        return carry

    lax.fori_loop(0, NPAIR, pair_body, 0)
    wait_out(0)
    wait_out(1)


@jax.jit
def kernel(suppressants, capacity, targets, randomness_source,
           possible_capacities, cum_probs):
    tgt_u8 = targets.astype(jnp.uint8).reshape(N // 128, 128)
    tbl = jnp.concatenate([
        possible_capacities.astype(jnp.float32),
        cum_probs.astype(jnp.float32),
        jnp.zeros((8,), jnp.float32),
    ])
    f32 = jnp.float32
    vbuf = lambda dt: (pltpu.VMEM((C,), dt), pltpu.VMEM((C,), dt))
    run = pl.kernel(
        _body,
        out_type=(jax.ShapeDtypeStruct((N,), f32),
                  jax.ShapeDtypeStruct((N,), f32)),
        mesh=plsc.VectorSubcoreMesh(core_axis_name="c", subcore_axis_name="s"),
        scratch_types=[
            vbuf(f32),
            vbuf(f32),
            vbuf(f32),
            vbuf(f32),
            (pltpu.VMEM((ROWS, 128), jnp.int32),
             pltpu.VMEM((ROWS, 128), jnp.int32)),
            vbuf(f32),
            vbuf(f32),
            pltpu.VMEM((16,), f32),
            (pltpu.SemaphoreType.DMA, pltpu.SemaphoreType.DMA),
            (pltpu.SemaphoreType.DMA, pltpu.SemaphoreType.DMA),
        ],
    )
    capacity_new, suppressants_new = run(
        suppressants, capacity, tgt_u8, randomness_source, tbl)
    return capacity_new, suppressants_new
